# baseline (speedup 1.0000x reference)
import functools

import jax
import jax.numpy as jnp
from jax import lax
from jax.experimental import pallas as pl
from jax.experimental.pallas import tpu as pltpu
from jax.experimental.pallas import tpu_sc as plsc

N_NODES = 10000
N_EDGES = 320000
INC = 128
OUTC = 128
KVOL = 27

NC = 2
NS = 16
NW = NC * NS
CH = 80
NCH = 128
NBUF = 4
NQ = 4
NCHQ = NCH // NQ
E_PAD = NW * NCH * CH
NPAD = 10112
RPT = NPAD // NS


def _mm_body(x_ref, w_ref, o_ref):
    o_ref[...] = jnp.dot(x_ref[...], w_ref[0],
                         preferred_element_type=jnp.float32)


def _matmul(x, W):
    n = x.shape[0]
    return pl.pallas_call(
        _mm_body,
        grid=(KVOL,),
        in_specs=[
            pl.BlockSpec((n, INC), lambda k: (0, 0)),
            pl.BlockSpec((1, INC, OUTC), lambda k: (k, 0, 0)),
        ],
        out_specs=pl.BlockSpec((n, OUTC), lambda k: (k, 0)),
        out_shape=jax.ShapeDtypeStruct((KVOL * n, OUTC), jnp.float32),
    )(x, W)


QSZ = (NCH // NQ) * CH


def _sc_body(h_hbm, zeros_hbm, gidx_hbm, didx_hbm, out_hbm,
             gv, dv, rows, gsems, ssems, acc):
    cid = lax.axis_index("c")
    sid = lax.axis_index("s")
    wid = sid * NC + cid
    pltpu.sync_copy(zeros_hbm, acc.at[pl.ds(sid * RPT, RPT)])
    plsc.subcore_barrier()

    @pl.loop(0, NQ)
    def _quarter(qi):
        pltpu.sync_copy(gidx_hbm.at[pl.ds((wid * NQ + qi) * QSZ, QSZ)], gv)
        pltpu.sync_copy(didx_hbm.at[wid * NQ + qi], dv)
        for b in range(NBUF):
            pltpu.async_copy(h_hbm.at[gv.at[pl.ds(b * CH, CH)]],
                             rows[b], gsems[b])

        @pl.loop(0, NCHQ, step=NBUF)
        def _chunk(ci):
            for b in range(NBUF):
                pltpu.make_async_copy(h_hbm.at[gv.at[pl.ds((ci + b) * CH, CH)]],
                                      rows[b], gsems[b]).wait()
                pltpu.async_copy(rows[b], acc.at[dv.at[ci + b]],
                                 ssems[b], add=True)
            for b in range(NBUF):
                pltpu.make_async_copy(rows[b], acc.at[dv.at[ci + b]],
                                      ssems[b]).wait()

                @pl.when(ci + NBUF + b < NCHQ)
                def _g():
                    pltpu.async_copy(
                        h_hbm.at[gv.at[pl.ds((ci + NBUF + b) * CH, CH)]],
                        rows[b], gsems[b])

    plsc.subcore_barrier()
    pltpu.sync_copy(acc.at[pl.ds(sid * RPT, RPT)],
                    out_hbm.at[cid, pl.ds(sid * RPT, RPT)])


@functools.cache
def _sc_scatter():
    return pl.kernel(
        _sc_body,
        out_type=jax.ShapeDtypeStruct((NC, NPAD, OUTC), jnp.float32),
        mesh=plsc.VectorSubcoreMesh(core_axis_name="c", subcore_axis_name="s"),
        scratch_types=[
            pltpu.VMEM((QSZ,), jnp.int32),
            pltpu.VMEM((NCHQ, CH), jnp.int32),
            [pltpu.VMEM((CH, OUTC), jnp.float32) for _ in range(NBUF)],
            [pltpu.SemaphoreType.DMA for _ in range(NBUF)],
            [pltpu.SemaphoreType.DMA for _ in range(NBUF)],
            pltpu.VMEM_SHARED((NPAD, OUTC), jnp.float32),
        ],
    )


E_BLK = 2560
N_BLK = E_PAD // E_BLK
N_VALID_BLK = N_EDGES // E_BLK


def _prep_body(src_ref, dst_ref, koff_ref, g_ref, d_ref):
    base = pl.program_id(0) * E_BLK
    ii = jax.lax.broadcasted_iota(jnp.int32, (1, E_BLK), 1) + base
    valid = ii < N_EDGES
    e_pad = ii - N_EDGES
    g_ref[...] = jnp.where(valid, koff_ref[0] * N_NODES + src_ref[0, 0], e_pad)
    d_ref[...] = jnp.where(valid, dst_ref[0, 0],
                           N_NODES + e_pad % (NPAD - N_NODES))


def _prep(edge_index, koff):
    clamp = lambda i: jnp.minimum(i, N_VALID_BLK - 1)
    return pl.pallas_call(
        _prep_body,
        grid=(N_BLK,),
        in_specs=[
            pl.BlockSpec((1, 1, 1, E_BLK), lambda i: (0, clamp(i), 0, 0)),
            pl.BlockSpec((1, 1, 1, E_BLK), lambda i: (1, clamp(i), 0, 0)),
            pl.BlockSpec((1, 1, E_BLK), lambda i: (clamp(i), 0, 0)),
        ],
        out_specs=[pl.BlockSpec((1, E_BLK), lambda i: (0, i)),
                   pl.BlockSpec((1, E_BLK), lambda i: (0, i))],
        out_shape=[jax.ShapeDtypeStruct((1, E_PAD), jnp.int32),
                   jax.ShapeDtypeStruct((1, E_PAD), jnp.int32)],
    )(edge_index.reshape(2, N_VALID_BLK, 1, E_BLK),
      edge_index.reshape(2, N_VALID_BLK, 1, E_BLK),
      koff.reshape(N_VALID_BLK, 1, E_BLK))


def _bn_body(p_ref, g_ref, b_ref, o_ref):
    n = o_ref.shape[0]
    s = p_ref[0, :n] + p_ref[1, :n]
    mean = jnp.sum(s, axis=0, keepdims=True) / n
    d = s - mean
    var = jnp.sum(d * d, axis=0, keepdims=True) / n
    y = d * lax.rsqrt(var + 1e-5) * g_ref[...] + b_ref[...]
    o_ref[...] = jnp.where(y >= 0, y, 0.01 * y)


def _bn(partials, gamma, beta, n):
    return pl.pallas_call(
        _bn_body,
        out_shape=jax.ShapeDtypeStruct((n, OUTC), jnp.float32),
    )(partials, gamma.reshape(1, OUTC), beta.reshape(1, OUTC))


def kernel(x, edge_index, koff, W, gamma, beta):
    n = x.shape[0]
    h = _matmul(x, W)
    gidx_f, didx_f = _prep(edge_index.astype(jnp.int32), koff.astype(jnp.int32))
    gidx_t = gidx_f.reshape(E_PAD)
    didx_t = didx_f.reshape(NW * NQ, NCHQ, CH)
    zeros = jnp.zeros((RPT, OUTC), jnp.float32)

    partials = _sc_scatter()(h, zeros, gidx_t, didx_t)
    return _bn(partials, gamma, beta, n)


# speedup vs baseline: 1.3488x; 1.3488x over previous
import functools

import jax
import jax.numpy as jnp
from jax import lax
from jax.experimental import pallas as pl
from jax.experimental.pallas import tpu as pltpu
from jax.experimental.pallas import tpu_sc as plsc

N_NODES = 10000
N_EDGES = 320000
INC = 128
OUTC = 128
KVOL = 27

NC = 2
NS = 16
NW = NC * NS
CH = 80
NCH = 128
NBUF = 4
NQ = 4
NCHQ = NCH // NQ
E_PAD = NW * NCH * CH
NPAD = 10112
RPT = NPAD // NS


def _mm_body(x_ref, w_ref, o_ref):
    o_ref[...] = jnp.dot(x_ref[...], w_ref[0],
                         preferred_element_type=jnp.float32)


def _matmul(x, W):
    n = x.shape[0]
    return pl.pallas_call(
        _mm_body,
        grid=(KVOL,),
        in_specs=[
            pl.BlockSpec((n, INC), lambda k: (0, 0)),
            pl.BlockSpec((1, INC, OUTC), lambda k: (k, 0, 0)),
        ],
        out_specs=pl.BlockSpec((n, OUTC), lambda k: (k, 0)),
        out_shape=jax.ShapeDtypeStruct((KVOL * n, OUTC), jnp.float32),
    )(x, W)


QSZ = (NCH // NQ) * CH


def _sc_body(h_hbm, zeros_hbm, gidx_hbm, didx_hbm, out_hbm,
             gv, dv, rows, gsems, ssems, acc):
    cid = lax.axis_index("c")
    sid = lax.axis_index("s")
    wid = sid * NC + cid
    pltpu.sync_copy(zeros_hbm, acc.at[pl.ds(sid * RPT, RPT)])
    plsc.subcore_barrier()

    @pl.loop(0, NQ)
    def _quarter(qi):
        pltpu.sync_copy(gidx_hbm.at[pl.ds((wid * NQ + qi) * QSZ, QSZ)], gv)
        pltpu.sync_copy(didx_hbm.at[wid * NQ + qi], dv)
        for b in range(NBUF):
            pltpu.async_copy(h_hbm.at[gv.at[pl.ds(b * CH, CH)]],
                             rows[b], gsems[b])

        @pl.loop(0, NCHQ, step=NBUF)
        def _chunk(ci):
            for b in range(NBUF):
                pltpu.make_async_copy(h_hbm.at[gv.at[pl.ds((ci + b) * CH, CH)]],
                                      rows[b], gsems[b]).wait()
                pltpu.async_copy(rows[b], acc.at[dv.at[ci + b]],
                                 ssems[b], add=True)
            for b in range(NBUF):
                pltpu.make_async_copy(rows[b], acc.at[dv.at[ci + b]],
                                      ssems[b]).wait()

                @pl.when(ci + NBUF + b < NCHQ)
                def _g():
                    pltpu.async_copy(
                        h_hbm.at[gv.at[pl.ds((ci + NBUF + b) * CH, CH)]],
                        rows[b], gsems[b])

    plsc.subcore_barrier()
    pltpu.sync_copy(acc.at[pl.ds(sid * RPT, RPT)],
                    out_hbm.at[cid, pl.ds(sid * RPT, RPT)])


@functools.cache
def _sc_scatter():
    return pl.kernel(
        _sc_body,
        out_type=jax.ShapeDtypeStruct((NC, NPAD, OUTC), jnp.float32),
        mesh=plsc.VectorSubcoreMesh(core_axis_name="c", subcore_axis_name="s"),
        scratch_types=[
            pltpu.VMEM((QSZ,), jnp.int32),
            pltpu.VMEM((NCHQ, CH), jnp.int32),
            [pltpu.VMEM((CH, OUTC), jnp.float32) for _ in range(NBUF)],
            [pltpu.SemaphoreType.DMA for _ in range(NBUF)],
            [pltpu.SemaphoreType.DMA for _ in range(NBUF)],
            pltpu.VMEM_SHARED((NPAD, OUTC), jnp.float32),
        ],
    )


def _bn_body(p_ref, g_ref, b_ref, o_ref):
    n = o_ref.shape[0]
    s = p_ref[0, :n] + p_ref[1, :n]
    mean = jnp.sum(s, axis=0, keepdims=True) / n
    d = s - mean
    var = jnp.sum(d * d, axis=0, keepdims=True) / n
    y = d * lax.rsqrt(var + 1e-5) * g_ref[...] + b_ref[...]
    o_ref[...] = jnp.where(y >= 0, y, 0.01 * y)


def _bn(partials, gamma, beta, n):
    return pl.pallas_call(
        _bn_body,
        out_shape=jax.ShapeDtypeStruct((n, OUTC), jnp.float32),
    )(partials, gamma.reshape(1, OUTC), beta.reshape(1, OUTC))


def kernel(x, edge_index, koff, W, gamma, beta):
    n = x.shape[0]
    h = _matmul(x, W)

    src = edge_index[0]
    dst = edge_index[1]
    gidx = (koff * n + src).astype(jnp.int32)
    pad = E_PAD - N_EDGES
    pad_ar = jnp.arange(pad, dtype=jnp.int32)
    gidx_t = jnp.concatenate([gidx, pad_ar])
    didx_t = jnp.concatenate(
        [dst.astype(jnp.int32), n + pad_ar % (NPAD - n)]).reshape(NW * NQ, NCHQ, CH)
    zeros = jnp.zeros((RPT, OUTC), jnp.float32)

    partials = _sc_scatter()(h, zeros, gidx_t, didx_t)
    return _bn(partials, gamma, beta, n)
